# baseline (speedup 1.0000x reference)
import numpy as np
import jax
import jax.numpy as jnp
from jax import lax
from jax.experimental import pallas as pl
from jax.experimental.pallas import tpu as pltpu
from jax.experimental.pallas import tpu_sc as plsc

_N = 4096
_B = 512
_NB = _N // _B
_PAIRS = ([(i, i) for i in range(_NB)]
          + [(i, j) for i in range(_NB) for j in range(i + 1, _NB)])
_P = len(_PAIRS)

_NW = 32
_ROWS_W = _N // _NW
_CH = 8
_NCHUNK = _ROWS_W // _CH


def _mud_sc_body(mu_hbm, out_hbm, mu_v, buf0, buf1, sem0, sem1):
    wid = lax.axis_index("s") * 2 + lax.axis_index("c")
    j0 = wid * _ROWS_W
    pltpu.sync_copy(mu_hbm, mu_v)
    bufs = (buf0, buf1)
    sems = (sem0, sem1)
    handles = []
    for c in range(_NCHUNK):
        b = c % 2
        if c >= 2:
            handles[c - 2].wait()

        base = (c * _CH // 16) * 16
        vec = mu_v[pl.ds(j0 + base, 16)]
        splats = []
        for r in range(_CH):
            idx = lax.iota(jnp.int32, 16) * 0 + (c * _CH - base + r)
            splats.append(vec.at[idx].get(mode="promise_in_bounds"))

        @plsc.parallel_loop(0, _N, step=16, unroll=4)
        def k_loop(k, b=b, splats=splats):
            chunk = mu_v[pl.ds(k, 16)]
            for r in range(_CH):
                bufs[b][pl.ds(r * _N + k, 16)] = splats[r] - chunk

        handles.append(pltpu.async_copy(
            bufs[b],
            out_hbm.at[pl.ds((j0 + c * _CH) * _N, _CH * _N)],
            sems[b]))
    handles[-2].wait()
    handles[-1].wait()


def _mud_sc(mu):
    mesh = plsc.VectorSubcoreMesh(core_axis_name="c", subcore_axis_name="s")
    return pl.kernel(
        _mud_sc_body,
        out_type=jax.ShapeDtypeStruct((_N * _N,), jnp.float32),
        mesh=mesh,
        scratch_types=[
            pltpu.VMEM((_N,), jnp.float32),
            pltpu.VMEM((_CH * _N,), jnp.float32),
            pltpu.VMEM((_CH * _N,), jnp.float32),
            pltpu.SemaphoreType.DMA,
            pltpu.SemaphoreType.DMA,
        ],
    )(mu)


def _main_body(im_ref, jm_ref, a_ref, sig_hbm, sd_hbm, diag_s, b_buf,
               sd_up, sd_lo, sems, bsems):
    t = pl.program_id(0)
    slot = jax.lax.rem(t, 2)
    ti = im_ref[t]
    tj = jm_ref[t]
    r0 = ti * _B
    c0 = tj * _B

    nch = _B // 128

    def b_fetch(step):
        sl = jax.lax.rem(step, 2)
        return pltpu.make_async_copy(
            sig_hbm.at[pl.ds(jm_ref[step] * _B, _B),
                       pl.ds(im_ref[step] * _B, _B)],
            b_buf.at[sl], bsems.at[sl])

    @pl.when(t == _NB - 2)
    def _prime_b0():
        b_fetch(_NB).start()

    @pl.when(t == _NB - 1)
    def _prime_b1():
        b_fetch(_NB + 1).start()

    def up_copies(sl, rr, cc):
        return [pltpu.make_async_copy(
            sd_up.at[sl, :, pl.ds(kk * 128, 128)],
            sd_hbm.at[pl.ds(rr, _B), cc // 128 + kk, :],
            sems.at[sl]) for kk in range(nch)]

    def lo_copies(sl, rr, cc):
        return [pltpu.make_async_copy(
            sd_lo.at[sl, :, pl.ds(kk * 128, 128)],
            sd_hbm.at[pl.ds(cc, _B), rr // 128 + kk, :],
            sems.at[sl]) for kk in range(nch)]

    def drain(step, sl):
        for c in up_copies(sl, r0, c0):
            c.wait()

        @pl.when(im_ref[step] != jm_ref[step])
        def _():
            for c in lo_copies(sl, r0, c0):
                c.wait()

    @pl.when(t >= 2)
    def _drain_prev():
        drain(t - 2, slot)

    a = a_ref[...]

    @pl.when(t < _NB)
    def _diag_step():
        rows = jax.lax.broadcasted_iota(jnp.int32, (_B, _B), 0)
        cols = jax.lax.broadcasted_iota(jnp.int32, (_B, _B), 1)
        dvec = jnp.sum(jnp.where(rows == cols, a, 0.0), axis=0)
        diag_s[0, pl.ds(r0, _B)] = dvec
        dsum = dvec[:, None] + dvec[None, :]
        sdv = jnp.sqrt(jnp.maximum(dsum - a - a.T, 1e-6))
        sd_up[slot] = sdv
        for c in up_copies(slot, r0, c0):
            c.start()

    @pl.when(t >= _NB)
    def _offdiag_step():
        b_fetch(t).wait()
        bt = b_buf[slot][...].T
        d_i = diag_s[0, pl.ds(r0, _B)]
        d_j = diag_s[0, pl.ds(c0, _B)]
        dsum = d_i[:, None] + d_j[None, :]
        sdv = jnp.sqrt(jnp.maximum(dsum - a - bt, 1e-6))
        sd_up[slot] = sdv
        sd_lo[slot] = sdv.T
        for c in up_copies(slot, r0, c0):
            c.start()
        for c in lo_copies(slot, r0, c0):
            c.start()

        @pl.when(t + 2 <= _P - 1)
        def _next_b():
            b_fetch(t + 2).start()

    @pl.when(t == _P - 1)
    def _drain_tail():
        drain(t - 1, 1 - slot)
        drain(t, slot)


def kernel(mu, Sigma):
    imap = jnp.asarray(np.array([p[0] for p in _PAIRS], dtype=np.int32))
    jmap = jnp.asarray(np.array([p[1] for p in _PAIRS], dtype=np.int32))

    grid_spec = pltpu.PrefetchScalarGridSpec(
        num_scalar_prefetch=2,
        grid=(_P,),
        in_specs=[
            pl.BlockSpec((_B, _B), lambda t, im, jm: (im[t], jm[t])),
            pl.BlockSpec(memory_space=pl.ANY),
        ],
        out_specs=[
            pl.BlockSpec(memory_space=pl.ANY),
        ],
        scratch_shapes=[
            pltpu.VMEM((1, _N), jnp.float32),
            pltpu.VMEM((2, _B, _B), jnp.float32),
            pltpu.VMEM((2, _B, _B), jnp.float32),
            pltpu.VMEM((2, _B, _B), jnp.float32),
            pltpu.SemaphoreType.DMA((2,)),
            pltpu.SemaphoreType.DMA((2,)),
        ],
    )
    sd = pl.pallas_call(
        _main_body,
        grid_spec=grid_spec,
        out_shape=[
            jax.ShapeDtypeStruct((_N, _N // 128, 128), jnp.float32),
        ],
    )(imap, jmap, Sigma, Sigma)[0]

    mud = _mud_sc(mu)
    return (mud, sd.reshape(_N * _N))


# speedup vs baseline: 1.0029x; 1.0029x over previous
import numpy as np
import jax
import jax.numpy as jnp
from jax import lax
from jax.experimental import pallas as pl
from jax.experimental.pallas import tpu as pltpu
from jax.experimental.pallas import tpu_sc as plsc

_N = 4096
_B = 512
_NB = _N // _B
_PAIRS = ([(i, i) for i in range(_NB)]
          + [(i, j) for i in range(_NB) for j in range(i + 1, _NB)])
_P = len(_PAIRS)

_NW = 32
_ROWS_W = _N // _NW
_CH = 8
_NCHUNK = _ROWS_W // _CH


def _mud_sc_body(mu_hbm, out_hbm, mu_v, buf0, buf1, sem0, sem1):
    wid = lax.axis_index("s") * 2 + lax.axis_index("c")
    j0 = wid * _ROWS_W
    pltpu.sync_copy(mu_hbm, mu_v)
    bufs = (buf0, buf1)
    sems = (sem0, sem1)
    handles = []
    for c in range(_NCHUNK):
        b = c % 2
        if c >= 2:
            handles[c - 2].wait()

        base = (c * _CH // 16) * 16
        vec = mu_v[pl.ds(j0 + base, 16)]
        splats = []
        for r in range(_CH):
            idx = lax.iota(jnp.int32, 16) * 0 + (c * _CH - base + r)
            splats.append(vec.at[idx].get(mode="promise_in_bounds"))

        @plsc.parallel_loop(0, _N, step=16, unroll=4)
        def k_loop(k, b=b, splats=splats):
            chunk = mu_v[pl.ds(k, 16)]
            for r in range(_CH):
                bufs[b][pl.ds(r * _N + k, 16)] = splats[r] - chunk

        handles.append(pltpu.async_copy(
            bufs[b],
            out_hbm.at[pl.ds((j0 + c * _CH) * _N, _CH * _N)],
            sems[b]))
    handles[-2].wait()
    handles[-1].wait()


def _mud_sc(mu):
    mesh = plsc.VectorSubcoreMesh(core_axis_name="c", subcore_axis_name="s")
    return pl.kernel(
        _mud_sc_body,
        out_type=jax.ShapeDtypeStruct((_N * _N,), jnp.float32),
        mesh=mesh,
        scratch_types=[
            pltpu.VMEM((_N,), jnp.float32),
            pltpu.VMEM((_CH * _N,), jnp.float32),
            pltpu.VMEM((_CH * _N,), jnp.float32),
            pltpu.SemaphoreType.DMA,
            pltpu.SemaphoreType.DMA,
        ],
    )(mu)


def _main_body(im_ref, jm_ref, a_ref, sig_hbm, sd_hbm, diag_s, b_buf,
               sd_up, sd_lo, sems, bsems):
    t = pl.program_id(0)
    slot = jax.lax.rem(t, 2)
    ti = im_ref[t]
    tj = jm_ref[t]
    r0 = ti * _B
    c0 = tj * _B

    nch = _B // 128

    def b_fetch(step):
        sl = jax.lax.rem(step, 3)
        return pltpu.make_async_copy(
            sig_hbm.at[pl.ds(jm_ref[step] * _B, _B),
                       pl.ds(im_ref[step] * _B, _B)],
            b_buf.at[sl], bsems.at[sl])

    @pl.when(t == _NB - 3)
    def _prime_b0():
        b_fetch(_NB).start()

    @pl.when(t == _NB - 2)
    def _prime_b1():
        b_fetch(_NB + 1).start()

    @pl.when(t == _NB - 1)
    def _prime_b2():
        b_fetch(_NB + 2).start()

    def up_copies(sl, rr, cc):
        return [pltpu.make_async_copy(
            sd_up.at[sl, :, pl.ds(kk * 128, 128)],
            sd_hbm.at[pl.ds(rr, _B), cc // 128 + kk, :],
            sems.at[sl]) for kk in range(nch)]

    def lo_copies(sl, rr, cc):
        return [pltpu.make_async_copy(
            sd_lo.at[sl, :, pl.ds(kk * 128, 128)],
            sd_hbm.at[pl.ds(cc, _B), rr // 128 + kk, :],
            sems.at[sl]) for kk in range(nch)]

    def drain(step, sl):
        for c in up_copies(sl, r0, c0):
            c.wait()

        @pl.when(im_ref[step] != jm_ref[step])
        def _():
            for c in lo_copies(sl, r0, c0):
                c.wait()

    @pl.when(t >= 2)
    def _drain_prev():
        drain(t - 2, slot)

    a = a_ref[...]

    @pl.when(t < _NB)
    def _diag_step():
        rows = jax.lax.broadcasted_iota(jnp.int32, (_B, _B), 0)
        cols = jax.lax.broadcasted_iota(jnp.int32, (_B, _B), 1)
        dvec = jnp.sum(jnp.where(rows == cols, a, 0.0), axis=0)
        diag_s[0, pl.ds(r0, _B)] = dvec
        dsum = dvec[:, None] + dvec[None, :]
        sdv = jnp.sqrt(jnp.maximum(dsum - a - a.T, 1e-6))
        sd_up[slot] = sdv
        for c in up_copies(slot, r0, c0):
            c.start()

    @pl.when(t >= _NB)
    def _offdiag_step():
        b_fetch(t).wait()
        bsl = jax.lax.rem(t, 3)
        bt = b_buf[bsl][...].T
        d_i = diag_s[0, pl.ds(r0, _B)]
        d_j = diag_s[0, pl.ds(c0, _B)]
        dsum = d_i[:, None] + d_j[None, :]
        sdv = jnp.sqrt(jnp.maximum(dsum - a - bt, 1e-6))
        sd_up[slot] = sdv
        sd_lo[slot] = sdv.T
        for c in up_copies(slot, r0, c0):
            c.start()
        for c in lo_copies(slot, r0, c0):
            c.start()

        @pl.when(t + 3 <= _P - 1)
        def _next_b():
            b_fetch(t + 3).start()

    @pl.when(t == _P - 1)
    def _drain_tail():
        drain(t - 1, 1 - slot)
        drain(t, slot)


def kernel(mu, Sigma):
    imap = jnp.asarray(np.array([p[0] for p in _PAIRS], dtype=np.int32))
    jmap = jnp.asarray(np.array([p[1] for p in _PAIRS], dtype=np.int32))

    grid_spec = pltpu.PrefetchScalarGridSpec(
        num_scalar_prefetch=2,
        grid=(_P,),
        in_specs=[
            pl.BlockSpec((_B, _B), lambda t, im, jm: (im[t], jm[t])),
            pl.BlockSpec(memory_space=pl.ANY),
        ],
        out_specs=[
            pl.BlockSpec(memory_space=pl.ANY),
        ],
        scratch_shapes=[
            pltpu.VMEM((1, _N), jnp.float32),
            pltpu.VMEM((3, _B, _B), jnp.float32),
            pltpu.VMEM((2, _B, _B), jnp.float32),
            pltpu.VMEM((2, _B, _B), jnp.float32),
            pltpu.SemaphoreType.DMA((2,)),
            pltpu.SemaphoreType.DMA((3,)),
        ],
    )
    sd = pl.pallas_call(
        _main_body,
        grid_spec=grid_spec,
        out_shape=[
            jax.ShapeDtypeStruct((_N, _N // 128, 128), jnp.float32),
        ],
    )(imap, jmap, Sigma, Sigma)[0]

    mud = _mud_sc(mu)
    return (mud, sd.reshape(_N * _N))
